# Initial kernel scaffold; baseline (speedup 1.0000x reference)
import jax
import jax.numpy as jnp
from jax.experimental import pallas as pl


def kernel(x, edge_index, W1, att_src1, att_dst1, b1, W2, att_src2, att_dst2, b2, Wm1, bm1, Wm2, bm2):
    raise NotImplementedError("write your pallas kernel here")


# speedup vs baseline: 23.4765x; 23.4765x over previous
import functools

import jax
import jax.numpy as jnp
from jax import lax
from jax.experimental import pallas as pl
from jax.experimental.pallas import tpu as pltpu
from jax.experimental.pallas import tpu_sc as plsc

N = 10000
NP = 10240
D = 128
C = 16
ROWW = 144
WCOL = 128
E = 320000
EL = E + N
CH = 64
EPAD = 331776
BLK = 1024
NBLK = NP // BLK
ZR = NP // 16

f32 = jnp.float32
i32 = jnp.int32


def _elu(v):
    return jnp.where(v > 0, v, jnp.exp(jnp.minimum(v, 0.0)) - 1.0)


def _row_tail(nrows):
    return (jnp.ones((nrows, 1), f32), jnp.zeros((nrows, ROWW - WCOL - 1), f32))


def _tc1_body(x_ref, w_ref, asrc_ref, adst_ref, hT_ref, as_ref, ad_ref):
    hb = jnp.dot(x_ref[...], w_ref[...], preferred_element_type=f32,
                 precision=lax.Precision.HIGHEST)
    one, pad = _row_tail(BLK)
    for h in range(2):
        hh = hb[:, h * D:(h + 1) * D]
        hT_ref[h] = jnp.concatenate([hh, one, pad], axis=1)
        as_ref[h, :] = jnp.sum(hh * asrc_ref[h][None, :], axis=1)
        ad_ref[h, :] = jnp.sum(hh * adst_ref[h][None, :], axis=1)


def _tc1(x_pad, W1, att_src, att_dst):
    return pl.pallas_call(
        _tc1_body,
        grid=(NBLK,),
        in_specs=[
            pl.BlockSpec((BLK, D), lambda i: (i, 0)),
            pl.BlockSpec((D, 2 * D), lambda i: (0, 0)),
            pl.BlockSpec((2, D), lambda i: (0, 0)),
            pl.BlockSpec((2, D), lambda i: (0, 0)),
        ],
        out_specs=[
            pl.BlockSpec((2, BLK, ROWW), lambda i: (0, i, 0)),
            pl.BlockSpec((2, BLK), lambda i: (0, i)),
            pl.BlockSpec((2, BLK), lambda i: (0, i)),
        ],
        out_shape=[
            jax.ShapeDtypeStruct((2, NP, ROWW), f32),
            jax.ShapeDtypeStruct((2, NP), f32),
            jax.ShapeDtypeStruct((2, NP), f32),
        ],
    )(x_pad, W1, att_src, att_dst)


def _tc2_body(acc_ref, b1_ref, w2_ref, asrc_ref, adst_ref,
              hT_ref, as_ref, ad_ref):
    b = b1_ref[...]
    v0 = acc_ref[0, :, 0:WCOL] / (acc_ref[0, :, WCOL:WCOL + 1] + 1e-30)
    v1 = acc_ref[1, :, 0:WCOL] / (acc_ref[1, :, WCOL:WCOL + 1] + 1e-30)
    x2 = jnp.concatenate([_elu(v0 + b[0, 0:D]), _elu(v1 + b[0, D:2 * D])],
                         axis=1)
    hb = jnp.dot(x2, w2_ref[...], preferred_element_type=f32,
                 precision=lax.Precision.HIGHEST)
    one, pad = _row_tail(BLK)
    hT_ref[0] = jnp.concatenate([hb, one, pad], axis=1)
    as_ref[0] = jnp.sum(hb * asrc_ref[0][None, :], axis=1)
    ad_ref[0] = jnp.sum(hb * adst_ref[0][None, :], axis=1)


def _tc2(acc1, b1r, W2, att_src, att_dst):
    return pl.pallas_call(
        _tc2_body,
        grid=(NBLK,),
        in_specs=[
            pl.BlockSpec((2, BLK, ROWW), lambda i: (0, i, 0)),
            pl.BlockSpec((1, 2 * D), lambda i: (0, 0)),
            pl.BlockSpec((2 * D, D), lambda i: (0, 0)),
            pl.BlockSpec((1, D), lambda i: (0, 0)),
            pl.BlockSpec((1, D), lambda i: (0, 0)),
        ],
        out_specs=[
            pl.BlockSpec((1, BLK, ROWW), lambda i: (0, i, 0)),
            pl.BlockSpec((1, BLK), lambda i: (0, i)),
            pl.BlockSpec((1, BLK), lambda i: (0, i)),
        ],
        out_shape=[
            jax.ShapeDtypeStruct((1, NP, ROWW), f32),
            jax.ShapeDtypeStruct((1, NP), f32),
            jax.ShapeDtypeStruct((1, NP), f32),
        ],
    )(acc1, b1r, W2, att_src, att_dst)


def _tc3_body(acc_ref, b2_ref, wm1_ref, bm1_ref, wm2_ref, bm2_ref, out_ref):
    num = acc_ref[0, :, 0:WCOL] + acc_ref[1, :, 0:WCOL]
    den = acc_ref[0, :, WCOL:WCOL + 1] + acc_ref[1, :, WCOL:WCOL + 1]
    h = _elu(num / (den + 1e-30) + b2_ref[0])
    m = jnp.maximum(
        jnp.dot(h, wm1_ref[...], preferred_element_type=f32,
                precision=lax.Precision.HIGHEST) + bm1_ref[0], 0.0)
    out_ref[...] = jnp.maximum(
        jnp.dot(m, wm2_ref[...], preferred_element_type=f32,
                precision=lax.Precision.HIGHEST) + bm2_ref[0], 0.0)


def _tc3(acc2, b2r, Wm1, bm1r, Wm2, bm2r):
    return pl.pallas_call(
        _tc3_body,
        grid=(NBLK,),
        in_specs=[
            pl.BlockSpec((2, BLK, ROWW), lambda i: (0, i, 0)),
            pl.BlockSpec((1, D), lambda i: (0, 0)),
            pl.BlockSpec((D, D), lambda i: (0, 0)),
            pl.BlockSpec((1, D), lambda i: (0, 0)),
            pl.BlockSpec((D, C), lambda i: (0, 0)),
            pl.BlockSpec((1, C), lambda i: (0, 0)),
        ],
        out_specs=pl.BlockSpec((BLK, C), lambda i: (i, 0)),
        out_shape=jax.ShapeDtypeStruct((NP, C), f32),
    )(acc2, b2r, Wm1, bm1r, Wm2, bm2r)


def _sc_edge(H):
    mesh = plsc.VectorSubcoreMesh(core_axis_name="c", subcore_axis_name="s")
    epg = EPAD // 16 if H == 2 else EPAD // 32
    nch = epg // CH

    @functools.partial(
        pl.kernel,
        out_type=jax.ShapeDtypeStruct((2 * NP, ROWW), f32),
        mesh=mesh,
        compiler_params=pltpu.CompilerParams(
            needs_layout_passes=False, use_tc_tiling_on_sc=False),
        scratch_types=[
            pltpu.VMEM((NP,), f32),
            pltpu.VMEM((NP,), f32),
            pltpu.VMEM((CH,), i32),
            pltpu.VMEM((CH,), i32),
            pltpu.VMEM((CH,), i32),
            pltpu.VMEM((CH, ROWW), f32),
            pltpu.VMEM_SHARED((NP, ROWW), f32),
            pltpu.SemaphoreType.DMA,
        ],
    )
    def k(hT, a_src, a_dst, srcp, dstp, zrows, acc_out,
          asv, adv, srcv, dstv, srcadj, buf, acc, sem):
        cc = lax.axis_index("c")
        ss = lax.axis_index("s")
        if H == 2:
            rowoff = cc * NP
            base = ss * epg
        else:
            rowoff = 0
            base = (cc * 16 + ss) * epg
        pltpu.sync_copy(a_src.at[pl.ds(rowoff, NP)], asv)
        pltpu.sync_copy(a_dst.at[pl.ds(rowoff, NP)], adv)
        pltpu.sync_copy(zrows, acc.at[pl.ds(ss * ZR, ZR)])
        plsc.subcore_barrier()

        lanes = lax.iota(i32, 16)

        def chunk(t, carry):
            off = base + t * CH
            pltpu.sync_copy(srcp.at[pl.ds(off, CH)], srcv)
            pltpu.sync_copy(dstp.at[pl.ds(off, CH)], dstv)
            if H == 2:
                for g in range(CH // 16):
                    sl = pl.ds(g * 16, 16)
                    srcadj[sl] = srcv[sl] + rowoff
            idxref = srcadj if H == 2 else srcv
            gat = pltpu.async_copy(hT.at[idxref], buf, sem)
            ws = []
            for g in range(CH // 16):
                sl = pl.ds(g * 16, 16)
                e = (plsc.load_gather(asv, [srcv[sl]])
                     + plsc.load_gather(adv, [dstv[sl]]))
                e = jnp.where(e >= 0.0, e, 0.2 * e)
                ws.append(jnp.exp(e))
            gat.wait()
            for g in range(CH // 16):
                for rl in range(16):
                    wk = jnp.full((16,), jnp.sum(
                        jnp.where(lanes == rl, ws[g], 0.0)))
                    r = g * 16 + rl
                    for j in range(ROWW // 16):
                        cs = pl.ds(j * 16, 16)
                        buf[r, cs] = buf[r, cs] * wk
            pltpu.sync_copy(buf, acc.at[dstv], add=True)
            return carry

        lax.fori_loop(0, nch, chunk, 0)
        plsc.subcore_barrier()
        pltpu.sync_copy(acc.at[pl.ds(ss * ZR, ZR)],
                        acc_out.at[pl.ds(cc * NP + ss * ZR, ZR)])

    return k


_SC2 = _sc_edge(2)
_SC1 = _sc_edge(1)


def kernel(x, edge_index, W1, att_src1, att_dst1, b1, W2, att_src2, att_dst2,
           b2, Wm1, bm1, Wm2, bm2):
    x_pad = jnp.pad(x, ((0, NP - N), (0, 0)))
    loop = jnp.arange(N, dtype=i32)
    padi = jnp.full((EPAD - EL,), N, i32)
    srcp = jnp.concatenate([edge_index[0].astype(i32), loop, padi])
    dstp = jnp.concatenate([edge_index[1].astype(i32), loop, padi])
    zrows = jnp.zeros((ZR, ROWW), f32)

    hT1, as1, ad1 = _tc1(x_pad, W1, att_src1, att_dst1)
    acc1 = _SC2(hT1.reshape(2 * NP, ROWW), as1.reshape(-1), ad1.reshape(-1),
                srcp, dstp, zrows).reshape(2, NP, ROWW)
    hT2, as2, ad2 = _tc2(acc1, b1.reshape(1, 2 * D), W2, att_src2, att_dst2)
    acc2 = _SC1(hT2.reshape(NP, ROWW), as2.reshape(-1), ad2.reshape(-1),
                srcp, dstp, zrows).reshape(2, NP, ROWW)
    out = _tc3(acc2, b2.reshape(1, D), Wm1, bm1.reshape(1, D),
               Wm2, bm2.reshape(1, C))
    return out[:N]
